# Initial kernel scaffold; baseline (speedup 1.0000x reference)
import jax
import jax.numpy as jnp
from jax.experimental import pallas as pl


def kernel(tile_cont, tile_disc, ent_cont, ent_disc, tile_table, ent_table, tile_wc, tile_bc, tile_wa, tile_wf, ent_wc, ent_bc, ent_wa, ent_wf):
    raise NotImplementedError("write your pallas kernel here")


# speedup vs baseline: 11.7373x; 11.7373x over previous
import functools

import jax
import jax.numpy as jnp
from jax import lax
from jax.experimental import pallas as pl
from jax.experimental.pallas import tpu as pltpu
from jax.experimental.pallas import tpu_sc as plsc

F32 = jnp.float32

B = 1024
N_TILE = 225
N_ENT = 100
VOCAB = 4096
EMBED = 64

NC = 2
NS = 16
NW = NC * NS
L = 16


def _tc_pre_body(tt, te, wet, wee, wcbt, wcbe, ct, ce,
                 t2t, t2e, mt, me, cct, cce):
    hi = lax.Precision.HIGHEST
    t2t[...] = jnp.dot(tt[...], wet[...], precision=hi)
    t2e[...] = jnp.dot(te[...], wee[...], precision=hi)
    mt[...] = jnp.dot(wcbt[...], wet[...], precision=hi)
    me[...] = jnp.dot(wcbe[...], wee[...], precision=hi)
    for cref, oref in ((ct, cct), (ce, cce)):
        c = cref[...]
        a = lax.broadcasted_iota(jnp.int32, c.shape, 0)
        oref[...] = jnp.where(a >= 2, c[:, :, 0:1] - c, c)


def _tc_pre(tt, te, wet, wee, wcbt, wcbe, ct, ce):
    out_shape = (
        jax.ShapeDtypeStruct((VOCAB, 128), F32),
        jax.ShapeDtypeStruct((VOCAB, 128), F32),
        jax.ShapeDtypeStruct((2, 128), F32),
        jax.ShapeDtypeStruct((2, 128), F32),
        jax.ShapeDtypeStruct((4, B, N_TILE), F32),
        jax.ShapeDtypeStruct((4, B, N_ENT), F32),
    )
    return pl.pallas_call(_tc_pre_body, out_shape=out_shape)(
        tt, te, wet, wee, wcbt, wcbe, ct, ce)


def _sc_branch(P, C, contc, disc, sd, t2, u, v0, ab):
    per_w = P // NW
    n_chunks = per_w // C
    n_grp = C // L
    assert per_w * NW == P and n_chunks * C == per_w and n_grp * L == C

    mesh = plsc.VectorSubcoreMesh(core_axis_name="c", subcore_axis_name="s")

    @functools.partial(
        pl.kernel, mesh=mesh,
        out_type=jax.ShapeDtypeStruct((P, EMBED), F32),
        compiler_params=pltpu.CompilerParams(
            needs_layout_passes=False, use_tc_tiling_on_sc=False),
        scratch_types=(
            [pltpu.VMEM((VOCAB,), F32)]
            + [pltpu.VMEM((EMBED,), F32)] * 2
            + [pltpu.VMEM((L,), F32)]
            + [pltpu.VMEM((per_w,), F32)] * 4
            + [pltpu.VMEM((per_w,), jnp.int32)] * 3
            + [pltpu.VMEM((C, EMBED), F32)] * 3
            + [pltpu.VMEM((C,), F32)] * 5
            + [pltpu.SemaphoreType.DMA]
        ),
    )
    def k(cont_hbm, disc_hbm, sd_hbm, t2_hbm, u_hbm, v0_hbm, ab_hbm, out_hbm,
          sd_v, u_v, v0_v, ab_v, c0_v, c1_v, c2_v, c3_v, d0_v, d1_v, d2_v,
          r0, r1, r2, pv0, pv1, pv2, pvw, pvs, sem):
        wid = lax.axis_index("s") * NC + lax.axis_index("c")
        base_w = wid * per_w
        pltpu.sync_copy(sd_hbm, sd_v)
        pltpu.sync_copy(u_hbm, u_v)
        pltpu.sync_copy(v0_hbm, v0_v)
        pltpu.sync_copy(ab_hbm, ab_v)
        cont_vs = (c0_v, c1_v, c2_v, c3_v)
        disc_vs = (d0_v, d1_v, d2_v)
        for a in range(4):
            pltpu.sync_copy(cont_hbm.at[pl.ds(a * P + base_w, per_w)],
                            cont_vs[a])
        for a in range(3):
            pltpu.sync_copy(disc_hbm.at[pl.ds(a * P + base_w, per_w)],
                            disc_vs[a])
        abv = ab_v[...]
        alpha = abv[0]
        beta = abv[1]
        uvec = [u_v[pl.ds(kk * L, L)] for kk in range(EMBED // L)]
        vvec = [v0_v[pl.ds(kk * L, L)] for kk in range(EMBED // L)]

        def chunk(ci, carry):
            o = ci * C
            cp0 = pltpu.async_copy(t2_hbm.at[d0_v.at[pl.ds(o, C)]], r0, sem)
            cp1 = pltpu.async_copy(t2_hbm.at[d1_v.at[pl.ds(o, C)]], r1, sem)
            cp2 = pltpu.async_copy(t2_hbm.at[d2_v.at[pl.ds(o, C)]], r2, sem)

            def grp(gi, c2):
                og = o + gi * L
                gl = gi * L
                c0 = c0_v[pl.ds(og, L)]
                c1 = c1_v[pl.ds(og, L)]
                cc2 = c2_v[pl.ds(og, L)]
                c3 = c3_v[pl.ds(og, L)]
                s0 = alpha * c0 + beta
                s1 = alpha * c1 + beta
                s2 = alpha * cc2 + beta
                s3 = alpha * c3 + beta
                s4 = plsc.load_gather(sd_v, [d0_v[pl.ds(og, L)]])
                s5 = plsc.load_gather(sd_v, [d1_v[pl.ds(og, L)]])
                s6 = plsc.load_gather(sd_v, [d2_v[pl.ds(og, L)]])
                m = jnp.maximum(
                    jnp.maximum(jnp.maximum(s0, s1), jnp.maximum(s2, s3)),
                    jnp.maximum(jnp.maximum(s4, s5), s6))
                e0 = jnp.exp(s0 - m)
                e1 = jnp.exp(s1 - m)
                e2 = jnp.exp(s2 - m)
                e3 = jnp.exp(s3 - m)
                e4 = jnp.exp(s4 - m)
                e5 = jnp.exp(s5 - m)
                e6 = jnp.exp(s6 - m)
                r = 1.0 / (((e0 + e1) + (e2 + e3)) + ((e4 + e5) + e6))
                p0 = e0 * r
                p1 = e1 * r
                p2 = e2 * r
                p3 = e3 * r
                pv0[pl.ds(gl, L)] = e4 * r
                pv1[pl.ds(gl, L)] = e5 * r
                pv2[pl.ds(gl, L)] = e6 * r
                pvw[pl.ds(gl, L)] = p0 * c0 + p1 * c1 + p2 * cc2 + p3 * c3
                pvs[pl.ds(gl, L)] = (p0 + p1) + (p2 + p3)
                return c2

            lax.fori_loop(0, n_grp, grp, 0, unroll=True)
            cp0.wait()
            cp1.wait()
            cp2.wait()

            def posg(gi, c2):
                gl = gi * L
                p4g = pv0[pl.ds(gl, L)]
                p5g = pv1[pl.ds(gl, L)]
                p6g = pv2[pl.ds(gl, L)]
                wg = pvw[pl.ds(gl, L)]
                psg = pvs[pl.ds(gl, L)]
                for j in range(L):
                    i = gl + j
                    p4 = p4g[j]
                    p5 = p5g[j]
                    p6 = p6g[j]
                    w = wg[j]
                    ps = psg[j]
                    for kk in range(EMBED // L):
                        sl = pl.ds(kk * L, L)
                        r0[i, sl] = (r0[i, sl] * p4 + r1[i, sl] * p5
                                     + r2[i, sl] * p6
                                     + uvec[kk] * w + vvec[kk] * ps)
                return c2

            lax.fori_loop(0, n_grp, posg, 0)
            pltpu.sync_copy(r0, out_hbm.at[pl.ds(base_w + o, C)])
            return carry

        lax.fori_loop(0, n_chunks, chunk, 0)

    return k(contc, disc, sd, t2, u, v0, ab)


def kernel(tile_cont, tile_disc, ent_cont, ent_disc, tile_table, ent_table,
           tile_wc, tile_bc, tile_wa, tile_wf,
           ent_wc, ent_bc, ent_wa, ent_wf):
    zpad = jnp.zeros((EMBED, 63), F32)
    wet = jnp.concatenate([tile_wf, tile_wa[:, None], zpad], axis=1)
    wee = jnp.concatenate([ent_wf, ent_wa[:, None], zpad], axis=1)
    wcbt = jnp.stack([tile_wc, tile_bc])
    wcbe = jnp.stack([ent_wc, ent_bc])
    ct = jnp.transpose(tile_cont, (2, 0, 1))
    ce = jnp.transpose(ent_cont, (2, 0, 1))

    t2t, t2e, mt, me, cct, cce = _tc_pre(
        tile_table, ent_table, wet, wee, wcbt, wcbe, ct, ce)

    outs = []
    for (t2x, mx, ccx, disc, n, c_chunk) in (
            (t2t, mt, cct, tile_disc, N_TILE, 96),
            (t2e, me, cce, ent_disc, N_ENT, 128)):
        p = B * n
        t2 = t2x[:, :EMBED]
        sd = t2x[:, EMBED]
        u = mx[0, :EMBED]
        v0 = mx[1, :EMBED]
        ab = jnp.concatenate([mx[:, EMBED], jnp.zeros((L - 2,), F32)])
        contc = ccx.reshape(4 * p)
        disc_t = jnp.transpose(disc, (2, 0, 1)).reshape(3 * p).astype(jnp.int32)
        out = _sc_branch(p, c_chunk, contc, disc_t, sd, t2, u, v0, ab)
        outs.append(out.reshape(B, n, EMBED))
    return (outs[0], outs[1])
